# baseline (speedup 1.0000x reference)
import functools

import jax
import jax.numpy as jnp
from jax import lax
from jax.experimental import pallas as pl
from jax.experimental.pallas import tpu as pltpu
from jax.experimental.pallas import tpu_sc as plsc

B = 4096
L = 200
D = 64
HID = 128
NUM_CLASSES = 2

NC = 2
NS = 16
NW = NC * NS
B_PER_W = B // NW
GROUPS = D // 16
ROW_UNROLL = 2

_mesh = plsc.VectorSubcoreMesh(core_axis_name="c", subcore_axis_name="s")


@functools.partial(
    pl.kernel,
    out_type=jax.ShapeDtypeStruct((B, D), jnp.float32),
    mesh=_mesh,
    compiler_params=pltpu.CompilerParams(use_tc_tiling_on_sc=False),
    scratch_types=[
        pltpu.VMEM((L, B_PER_W), jnp.int32),
        pltpu.VMEM((B_PER_W, D), jnp.float32),
        pltpu.VMEM((B_PER_W, D), jnp.float32),
        pltpu.VMEM((B_PER_W, D), jnp.float32),
        pltpu.SemaphoreType.DMA,
        pltpu.SemaphoreType.DMA,
    ],
)
def _pool(xt_hbm, emb_hbm, out_hbm, idx_v, buf0, buf1, acc_v, sem0, sem1):
    wid = lax.axis_index("s") * NC + lax.axis_index("c")
    base = wid * B_PER_W

    pltpu.sync_copy(xt_hbm.at[:, pl.ds(base, B_PER_W)], idx_v)

    bufs = (buf0, buf1)
    sems = (sem0, sem1)

    def start(pos, half):
        pltpu.make_async_copy(
            emb_hbm.at[idx_v.at[pos]], bufs[half], sems[half]
        ).start()

    def wait(pos, half):
        pltpu.make_async_copy(
            emb_hbm.at[idx_v.at[pos]], bufs[half], sems[half]
        ).wait()

    zero = jnp.zeros((16,), jnp.float32)

    def zero_body(r, _):
        for u in range(ROW_UNROLL):
            row = r * ROW_UNROLL + u
            for g in range(GROUPS):
                acc_v[row, pl.ds(g * 16, 16)] = zero
        return 0

    lax.fori_loop(0, B_PER_W // ROW_UNROLL, zero_body, 0)

    start(0, 0)
    start(1, 1)

    def accum(buf):
        def row_body(r, _):
            for u in range(ROW_UNROLL):
                row = r * ROW_UNROLL + u
                for g in range(GROUPS):
                    plsc.addupdate(
                        acc_v.at[row, pl.ds(g * 16, 16)],
                        buf[row, pl.ds(g * 16, 16)],
                    )
            return 0

        lax.fori_loop(0, B_PER_W // ROW_UNROLL, row_body, 0)

    def pos_body(i, _):
        for half in range(2):
            pos = i * 2 + half
            wait(pos, half)
            accum(bufs[half])

            @pl.when(i < L // 2 - 1)
            def _():
                start(pos + 2, half)

        return 0

    lax.fori_loop(0, L // 2, pos_body, 0)

    pltpu.sync_copy(acc_v, out_hbm.at[pl.ds(base, B_PER_W)])


def _mlp_body(m_ref, vw_ref, vb_ref, ww_ref, wb_ref, out_ref):
    m = m_ref[...]
    h = jnp.dot(m, vw_ref[...], preferred_element_type=jnp.float32)
    h = jnp.maximum(h + vb_ref[...], 0.0)
    logits = jnp.dot(h, ww_ref[...], preferred_element_type=jnp.float32)
    logits = logits + wb_ref[...]
    mx = jnp.max(logits, axis=1, keepdims=True)
    s = logits - mx
    lse = jnp.log(jnp.sum(jnp.exp(s), axis=1, keepdims=True))
    out_ref[...] = s - lse


def _mlp(m, vw_t, vb, ww_t, wb):
    return pl.pallas_call(
        _mlp_body,
        out_shape=jax.ShapeDtypeStruct((B, NUM_CLASSES), jnp.float32),
    )(m, vw_t, vb, ww_t, wb)


@jax.jit
def kernel(x, emb, V_w, V_b, W_w, W_b):
    xt = x.astype(jnp.int32).T
    m_sum = _pool(xt, emb)
    vw_t = V_w.T * jnp.float32(1.0 / L)
    return _mlp(m_sum, vw_t, V_b.reshape(1, HID), W_w.T, W_b.reshape(1, NUM_CLASSES))
